# baseline (speedup 1.0000x reference)
import jax
import jax.numpy as jnp
from jax import lax
from jax.experimental import pallas as pl
from jax.experimental.pallas import tpu as pltpu

_EPS = 1e-5
_BT = 2


def _build_body(h, w, c, bt):
    hw = h * w

    def _conv_in(img, w_ref, p_ref):
        p_ref[:, c:2 * c] = img
        p_ref[w:, 0:c] = img[:hw - w]
        p_ref[:w, 0:c] = img[w:2 * w]
        p_ref[:hw - w, 2 * c:3 * c] = img[w:]
        p_ref[hw - w:, 2 * c:3 * c] = img[hw - 2 * w:hw - w]

        d = jnp.dot(p_ref[...], w_ref[...],
                    preferred_element_type=jnp.float32)

        d3 = d.reshape(h, w, 3 * c)
        d0 = d3[:, :, 0:c]
        d1 = d3[:, :, c:2 * c]
        d2 = d3[:, :, 2 * c:3 * c]
        s0 = jnp.concatenate([d0[:, 1:2], d0[:, 0:w - 1]], axis=1)
        s2 = jnp.concatenate([d2[:, 1:w], d2[:, w - 2:w - 1]], axis=1)
        acc = d1 + s0 + s2

        inv_hw = 1.0 / hw
        mean = jnp.sum(acc, axis=(0, 1), keepdims=True) * inv_hw
        msq = jnp.sum(acc * acc, axis=(0, 1), keepdims=True) * inv_hw
        var = msq - mean * mean
        scale = lax.rsqrt(var + _EPS)
        return ((acc - mean) * scale).reshape(hw, c)

    def _body(x_ref, w1_ref, w2_ref, o_ref, *p_refs):
        for i in range(bt):
            xm = x_ref[i].reshape(c, hw)
            x = jnp.transpose(xm.astype(jnp.bfloat16), (1, 0))
            y = jnp.maximum(_conv_in(x, w1_ref, p_refs[i]),
                            0.0).astype(jnp.bfloat16)
            z = _conv_in(y, w2_ref, p_refs[i])
            o_ref[i] = (xm + jnp.transpose(z, (1, 0))).reshape(c, h, w)

    return _body


def _resnet_block(x_nchw, w1, w2):
    n, c, h, w = x_nchw.shape
    hw = h * w
    bt = _BT if n % _BT == 0 else 1

    w1f = jnp.transpose(w1, (0, 2, 1, 3)).reshape(3 * c, 3 * c)
    w1f = w1f.astype(jnp.bfloat16)
    w2f = jnp.transpose(w2, (0, 2, 1, 3)).reshape(3 * c, 3 * c)
    w2f = w2f.astype(jnp.bfloat16)

    out = pl.pallas_call(
        _build_body(h, w, c, bt),
        out_shape=jax.ShapeDtypeStruct((n, c, h, w), jnp.float32),
        grid=(n // bt,),
        in_specs=[
            pl.BlockSpec((bt, c, h, w), lambda b: (b, 0, 0, 0)),
            pl.BlockSpec((3 * c, 3 * c), lambda b: (0, 0)),
            pl.BlockSpec((3 * c, 3 * c), lambda b: (0, 0)),
        ],
        out_specs=pl.BlockSpec((bt, c, h, w), lambda b: (b, 0, 0, 0)),
        scratch_shapes=[pltpu.VMEM((hw, 3 * c), jnp.bfloat16)
                        for _ in range(bt)],
        compiler_params=pltpu.CompilerParams(
            dimension_semantics=("parallel",),
            vmem_limit_bytes=56 * 1024 * 1024,
        ),
    )(x_nchw, w1f, w2f)

    return out


@jax.jit
def kernel(x_nchw, w1, b1, w2, b2):
    del b1, b2
    return _resnet_block(x_nchw, w1, w2)


# speedup vs baseline: 1.8251x; 1.8251x over previous
import jax
import jax.numpy as jnp
from jax import lax
from jax.experimental import pallas as pl
from jax.experimental.pallas import tpu as pltpu

_EPS = 1e-5
_BT = 4


def _build_body(h, w, c, bt):
    hw = h * w

    def _conv_in(img, w_ref, p_ref):
        p_ref[:, c:2 * c] = img
        p_ref[w:, 0:c] = img[:hw - w]
        p_ref[:w, 0:c] = img[w:2 * w]
        p_ref[:hw - w, 2 * c:3 * c] = img[w:]
        p_ref[hw - w:, 2 * c:3 * c] = img[hw - 2 * w:hw - w]

        d = jnp.dot(p_ref[...], w_ref[...],
                    preferred_element_type=jnp.float32)

        d3 = d.reshape(h, w, 3 * c)
        d0 = d3[:, :, 0:c]
        d1 = d3[:, :, c:2 * c]
        d2 = d3[:, :, 2 * c:3 * c]
        s0 = jnp.concatenate([d0[:, 1:2], d0[:, 0:w - 1]], axis=1)
        s2 = jnp.concatenate([d2[:, 1:w], d2[:, w - 2:w - 1]], axis=1)
        acc = d1 + s0 + s2

        inv_hw = 1.0 / hw
        mean = jnp.sum(acc, axis=(0, 1), keepdims=True) * inv_hw
        msq = jnp.sum(acc * acc, axis=(0, 1), keepdims=True) * inv_hw
        var = msq - mean * mean
        scale = lax.rsqrt(var + _EPS)
        return ((acc - mean) * scale).reshape(hw, c)

    def _body(x_ref, w1_ref, w2_ref, o_ref, *p_refs):
        for i in range(bt):
            x = x_ref[i]
            y = jnp.maximum(_conv_in(x, w1_ref, p_refs[i]),
                            0.0).astype(jnp.bfloat16)
            z = _conv_in(y, w2_ref, p_refs[i])
            o_ref[i] = (x.astype(jnp.float32) + z).astype(jnp.bfloat16)

    return _body


def _resnet_block(x_nchw, w1, w2):
    n, c, h, w = x_nchw.shape
    hw = h * w
    bt = _BT if n % _BT == 0 else 1

    xt = jnp.transpose(x_nchw, (0, 2, 3, 1)).reshape(n, hw, c)
    xt = xt.astype(jnp.bfloat16)

    w1f = jnp.transpose(w1, (0, 2, 1, 3)).reshape(3 * c, 3 * c)
    w1f = w1f.astype(jnp.bfloat16)
    w2f = jnp.transpose(w2, (0, 2, 1, 3)).reshape(3 * c, 3 * c)
    w2f = w2f.astype(jnp.bfloat16)

    out = pl.pallas_call(
        _build_body(h, w, c, bt),
        out_shape=jax.ShapeDtypeStruct((n, hw, c), jnp.bfloat16),
        grid=(n // bt,),
        in_specs=[
            pl.BlockSpec((bt, hw, c), lambda b: (b, 0, 0)),
            pl.BlockSpec((3 * c, 3 * c), lambda b: (0, 0)),
            pl.BlockSpec((3 * c, 3 * c), lambda b: (0, 0)),
        ],
        out_specs=pl.BlockSpec((bt, hw, c), lambda b: (b, 0, 0)),
        scratch_shapes=[pltpu.VMEM((hw, 3 * c), jnp.bfloat16)
                        for _ in range(bt)],
        compiler_params=pltpu.CompilerParams(
            dimension_semantics=("parallel",),
            vmem_limit_bytes=56 * 1024 * 1024,
        ),
    )(xt, w1f, w2f)

    out = jnp.transpose(out.reshape(n, h, w, c), (0, 3, 1, 2))
    return out.astype(jnp.float32)


@jax.jit
def kernel(x_nchw, w1, b1, w2, b2):
    del b1, b2
    return _resnet_block(x_nchw, w1, w2)


# speedup vs baseline: 2.1043x; 1.1530x over previous
import jax
import jax.numpy as jnp
from jax import lax
from jax.experimental import pallas as pl
from jax.experimental.pallas import tpu as pltpu

_EPS = 1e-5
_BT = 4


def _build_body(h, w, c, bt):
    hw = h * w

    def _conv_in(img, w_ref, p_ref):
        p_ref[:, c:2 * c] = img
        p_ref[w:, 0:c] = img[:hw - w]
        p_ref[:w, 0:c] = img[w:2 * w]
        p_ref[:hw - w, 2 * c:3 * c] = img[w:]
        p_ref[hw - w:, 2 * c:3 * c] = img[hw - 2 * w:hw - w]

        d = jnp.dot(p_ref[...], w_ref[...],
                    preferred_element_type=jnp.float32)

        d3 = d.reshape(h, w, 3 * c)
        d0 = d3[:, :, 0:c]
        d1 = d3[:, :, c:2 * c]
        d2 = d3[:, :, 2 * c:3 * c]
        s0 = jnp.concatenate([d0[:, 1:2], d0[:, 0:w - 1]], axis=1)
        s2 = jnp.concatenate([d2[:, 1:w], d2[:, w - 2:w - 1]], axis=1)
        acc = d1 + s0 + s2

        inv_hw = 1.0 / hw
        mean = jnp.sum(acc, axis=(0, 1), keepdims=True) * inv_hw
        msq = jnp.sum(acc * acc, axis=(0, 1), keepdims=True) * inv_hw
        var = msq - mean * mean
        scale = lax.rsqrt(var + _EPS)
        return ((acc - mean) * scale).reshape(hw, c)

    def _body(x_ref, w1_ref, w2_ref, o_ref, *p_refs):
        for i in range(bt):
            x = x_ref[i]
            y = jnp.maximum(_conv_in(x, w1_ref, p_refs[i]),
                            0.0).astype(jnp.bfloat16)
            z = _conv_in(y, w2_ref, p_refs[i])
            o_ref[i] = x.astype(jnp.float32) + z

    return _body


def _resnet_block(x_nchw, w1, w2):
    n, c, h, w = x_nchw.shape
    hw = h * w
    bt = _BT if n % _BT == 0 else 1

    xt = jnp.transpose(x_nchw, (0, 2, 3, 1)).reshape(n, hw, c)
    xt = xt.astype(jnp.bfloat16)

    w1f = jnp.transpose(w1, (0, 2, 1, 3)).reshape(3 * c, 3 * c)
    w1f = w1f.astype(jnp.bfloat16)
    w2f = jnp.transpose(w2, (0, 2, 1, 3)).reshape(3 * c, 3 * c)
    w2f = w2f.astype(jnp.bfloat16)

    out = pl.pallas_call(
        _build_body(h, w, c, bt),
        out_shape=jax.ShapeDtypeStruct((n, hw, c), jnp.float32),
        grid=(n // bt,),
        in_specs=[
            pl.BlockSpec((bt, hw, c), lambda b: (b, 0, 0)),
            pl.BlockSpec((3 * c, 3 * c), lambda b: (0, 0)),
            pl.BlockSpec((3 * c, 3 * c), lambda b: (0, 0)),
        ],
        out_specs=pl.BlockSpec((bt, hw, c), lambda b: (b, 0, 0)),
        scratch_shapes=[pltpu.VMEM((hw, 3 * c), jnp.bfloat16)
                        for _ in range(bt)],
        compiler_params=pltpu.CompilerParams(
            dimension_semantics=("parallel",),
            vmem_limit_bytes=56 * 1024 * 1024,
        ),
    )(xt, w1f, w2f)

    return jnp.transpose(out.reshape(n, h, w, c), (0, 3, 1, 2))


@jax.jit
def kernel(x_nchw, w1, b1, w2, b2):
    del b1, b2
    return _resnet_block(x_nchw, w1, w2)
